# Initial kernel scaffold; baseline (speedup 1.0000x reference)
import jax
import jax.numpy as jnp
from jax.experimental import pallas as pl


def kernel(x, w1, w2):
    raise NotImplementedError("write your pallas kernel here")


# speedup vs baseline: 1.0003x; 1.0003x over previous
import functools

import jax
import jax.numpy as jnp
from jax.experimental import pallas as pl
from jax.experimental.pallas import tpu as pltpu

_VMEM_LIMIT = 40 * 1024 * 1024


def _se_kernel(x_ref, w1_ref, w2_ref, o_ref, *, inv_l):
    xs = x_ref[...]
    pooled = jnp.sum(xs, axis=-1, dtype=jnp.float32) * inv_l
    h = jax.lax.dot_general(
        pooled, w1_ref[...], (((1,), (1,)), ((), ())),
        preferred_element_type=jnp.float32)
    h = jnp.maximum(h, 0.0)
    g = jax.lax.dot_general(
        h, w2_ref[...], (((1,), (1,)), ((), ())),
        preferred_element_type=jnp.float32)
    g = jax.nn.sigmoid(g)
    o_ref[...] = xs * g.astype(o_ref.dtype)[:, :, None]


@functools.partial(jax.jit, static_argnames=("tb",))
def _se_call(x, w1, w2, tb):
    B, C, L = x.shape
    Cr = w1.shape[0]
    body = functools.partial(_se_kernel, inv_l=1.0 / L)
    return pl.pallas_call(
        body,
        out_shape=jax.ShapeDtypeStruct((B, C, L), x.dtype),
        grid=(B // tb,),
        in_specs=[
            pl.BlockSpec((tb, C, L), lambda b: (b, 0, 0)),
            pl.BlockSpec((Cr, C), lambda b: (0, 0)),
            pl.BlockSpec((C, Cr), lambda b: (0, 0)),
        ],
        out_specs=pl.BlockSpec((tb, C, L), lambda b: (b, 0, 0)),
        compiler_params=pltpu.CompilerParams(
            dimension_semantics=("parallel",),
            vmem_limit_bytes=_VMEM_LIMIT,
        ),
    )(x, w1, w2)


def kernel(x, w1, w2):
    B, C, L = x.shape
    itemsize = jnp.dtype(x.dtype).itemsize
    tb = 1
    for d in range(B, 0, -1):
        if B % d == 0 and 4 * d * C * L * itemsize + 2**21 <= _VMEM_LIMIT:
            tb = d
            break
    return _se_call(x, w1, w2, tb)
